# baseline (speedup 1.0000x reference)
import functools

import jax
import jax.numpy as jnp
from jax.experimental import pallas as pl
from jax.experimental.pallas import tpu as pltpu

_LANE = 128


def _fused_kernel(idx_ref, w1_hbm, w2_ref, out_ref, hid_ref, sem,
                  *, bsz, s_chunks):
    for b in range(bsz):
        pltpu.make_async_copy(
            w1_hbm.at[idx_ref[b]], hid_ref.at[b], sem).start()
    for b in range(bsz):
        pltpu.make_async_copy(
            w1_hbm.at[idx_ref[0]], hid_ref.at[0], sem).wait()

    chunks = [hid_ref[:, s, :] for s in range(s_chunks)]
    h = chunks[0] if s_chunks == 1 else jnp.concatenate(chunks, axis=1)
    out_ref[...] = jnp.dot(h, w2_ref[...],
                           preferred_element_type=jnp.float32)


def kernel(idx, w1, w2):
    (bsz,) = idx.shape
    voc, emb = w1.shape
    assert w2.shape == (emb, voc) and emb % _LANE == 0
    s_chunks = emb // _LANE
    w1_rows = w1.reshape(voc, s_chunks, _LANE)
    tn = voc // 2

    grid_spec = pltpu.PrefetchScalarGridSpec(
        num_scalar_prefetch=1,
        grid=(2,),
        in_specs=[
            pl.BlockSpec(memory_space=pl.ANY),
            pl.BlockSpec((emb, tn), lambda c, idx_ref: (0, c)),
        ],
        out_specs=pl.BlockSpec((bsz, tn), lambda c, idx_ref: (0, c)),
        scratch_shapes=[
            pltpu.VMEM((bsz, s_chunks, _LANE), jnp.float32),
            pltpu.SemaphoreType.DMA,
        ],
    )
    return pl.pallas_call(
        functools.partial(_fused_kernel, bsz=bsz, s_chunks=s_chunks),
        grid_spec=grid_spec,
        out_shape=jax.ShapeDtypeStruct((bsz, voc), jnp.float32),
        compiler_params=pltpu.CompilerParams(
            dimension_semantics=("parallel",),
            disable_bounds_checks=True,
        ),
    )(idx, w1_rows, w2)
